# Initial kernel scaffold; baseline (speedup 1.0000x reference)
import jax
import jax.numpy as jnp
from jax.experimental import pallas as pl


def kernel(heatmap, boxes, classes):
    raise NotImplementedError("write your pallas kernel here")


# speedup vs baseline: 3.7975x; 3.7975x over previous
import functools

import jax
import jax.numpy as jnp
from jax import lax
from jax.experimental import pallas as pl
from jax.experimental.pallas import tpu as pltpu
from jax.experimental.pallas import tpu_sc as plsc

_C, _W, _H = 80, 128, 128
_N = 1000
_NPAD = 1024
_NTILES = 32
_R = 4
_MAXC = 3
_L = 16


def _splat_kernel(heat_hbm, boxes_hbm, classes_hbm, out_hbm, buf, boxes_v, cls_v):
    wid = lax.axis_index("s") * 2 + lax.axis_index("c")
    c0 = (wid * _C) // _NTILES
    c1 = ((wid + 1) * _C) // _NTILES

    pltpu.sync_copy(boxes_hbm, boxes_v)
    pltpu.sync_copy(classes_hbm, cls_v)
    pltpu.sync_copy(heat_hbm.at[pl.ds(c0, _MAXC)], buf)

    lanes = lax.iota(jnp.int32, _L)

    def body(i, carry):
        c = cls_v[pl.ds(i, _L)][0]

        @pl.when(jnp.logical_and(c >= c0, c < c1))
        def _():
            bv = boxes_v[pl.ds(4 * i, _L)]
            cx = (bv[0] + bv[2]) // 2
            cy = (bv[1] + bv[3]) // 2
            lc = c - c0
            y0 = jnp.clip(cy - _R, 0, _H - _L)
            dy = (lanes + (y0 - cy)).astype(jnp.float32)
            ney2 = -(dy * dy)
            for r in range(2 * _R + 1):
                x = jnp.clip(cx + (r - _R), 0, _W - 1)
                dxf = (x - cx).astype(jnp.float32)
                g = jnp.exp(ney2 - dxf * dxf)
                old = buf[lc, x, pl.ds(y0, _L)]
                buf[lc, x, pl.ds(y0, _L)] = jnp.maximum(old, g)

        return carry

    lax.fori_loop(0, _N, body, 0)

    pltpu.sync_copy(buf.at[pl.ds(0, 2)], out_hbm.at[pl.ds(c0, 2)])

    @pl.when(c1 - c0 == 3)
    def _():
        pltpu.sync_copy(buf.at[2], out_hbm.at[c0 + 2])


def kernel(heatmap, boxes, classes):
    boxes = boxes.astype(jnp.int32)
    classes = classes.astype(jnp.int32)
    boxes_flat = jnp.zeros((4 * _NPAD,), jnp.int32).at[: 4 * _N].set(
        boxes.reshape(-1)
    )
    cls_pad = jnp.full((_NPAD,), _C + 1, jnp.int32).at[:_N].set(classes)

    mesh = plsc.VectorSubcoreMesh(
        core_axis_name="c", subcore_axis_name="s", num_cores=2, num_subcores=16
    )
    run = functools.partial(
        pl.kernel,
        out_type=jax.ShapeDtypeStruct((_C, _W, _H), jnp.float32),
        mesh=mesh,
        scratch_types=[
            pltpu.VMEM((_MAXC, _W, _H), jnp.float32),
            pltpu.VMEM((4 * _NPAD,), jnp.int32),
            pltpu.VMEM((_NPAD,), jnp.int32),
        ],
    )(_splat_kernel)
    return run(heatmap, boxes_flat, cls_pad)
